# baseline (speedup 1.0000x reference)
import functools

import jax
import jax.numpy as jnp
from jax import lax
from jax.experimental import pallas as pl
from jax.experimental.pallas import tpu as pltpu
from jax.experimental.pallas import tpu_sc as plsc

N = 10000
E = 320000
D = 128
NG = 64
NCLS = 10

NC = 2
NS = 16
NW = NC * NS
CB = 128
NCH = 80
EP = NW * NCH * CB
NPAD = 10112
RPT = NPAD // NS
NROW = 80


@functools.lru_cache(maxsize=1)
def _sc_kernels():
    mesh = plsc.VectorSubcoreMesh(
        core_axis_name="c", subcore_axis_name="s",
        num_cores=NC, num_subcores=NS)

    @functools.partial(
        pl.kernel,
        out_type=jax.ShapeDtypeStruct((NC, NPAD, D), jnp.float32),
        mesh=mesh,
        scratch_types=[
            pltpu.VMEM((NCH, CB), jnp.int32),
            pltpu.VMEM((CB, D), jnp.float32),
            pltpu.VMEM_SHARED((NPAD, D), jnp.float32),
            pltpu.SemaphoreType.DMA,
        ],
    )
    def deg_kernel(dst_hbm, zeros_hbm, ones_hbm, out_hbm, didx, ones_v, deg_sh,
                   sem):
        c = lax.axis_index("c")
        s = lax.axis_index("s")
        wid = c * NS + s
        lo = s * RPT
        pltpu.sync_copy(dst_hbm.at[wid], didx)
        pltpu.sync_copy(zeros_hbm.at[pl.ds(lo, RPT)], deg_sh.at[pl.ds(lo, RPT)])
        pltpu.sync_copy(ones_hbm, ones_v)
        plsc.subcore_barrier()

        @pl.loop(0, NCH // 8)
        def _(g):
            for b in range(8):
                pltpu.async_copy(
                    ones_v, deg_sh.at[didx.at[g * 8 + b]], sem, add=True)
            for b in range(8):
                pltpu.make_async_copy(ones_hbm, ones_v, sem).wait()

        plsc.subcore_barrier()
        pltpu.sync_copy(deg_sh.at[pl.ds(lo, RPT)],
                        out_hbm.at[c].at[pl.ds(lo, RPT)])

    @functools.partial(
        pl.kernel,
        out_type=jax.ShapeDtypeStruct((NC, NPAD, D), jnp.float32),
        mesh=mesh,
        scratch_types=[
            pltpu.VMEM((CB,), jnp.int32),
            pltpu.VMEM((CB,), jnp.int32),
            pltpu.VMEM((NCH, CB), jnp.int32),
            pltpu.VMEM((CB, D), jnp.float32),
            pltpu.VMEM((CB, D), jnp.float32),
            pltpu.VMEM_SHARED((NPAD, D), jnp.float32),
            pltpu.SemaphoreType.DMA,
            pltpu.SemaphoreType.DMA,
            pltpu.SemaphoreType.DMA,
            pltpu.SemaphoreType.DMA,
        ],
    )
    def agg_kernel(hp_hbm, src_hbm, dst_hbm, zeros_hbm, out_hbm,
                   sidx0, sidx1, didx, rows0, rows1,
                   agg_sh, gsem0, gsem1, isem0, isem1):
        c = lax.axis_index("c")
        s = lax.axis_index("s")
        wid = c * NS + s
        lo = s * RPT
        sidx = (sidx0, sidx1)
        rows = (rows0, rows1)
        gsem = (gsem0, gsem1)
        isem = (isem0, isem1)

        pltpu.sync_copy(dst_hbm.at[wid], didx)
        pltpu.sync_copy(zeros_hbm.at[pl.ds(lo, RPT)], agg_sh.at[pl.ds(lo, RPT)])
        plsc.subcore_barrier()

        pltpu.sync_copy(src_hbm.at[wid].at[0], sidx0)
        pltpu.async_copy(hp_hbm.at[sidx0], rows0, gsem0)
        pltpu.async_copy(src_hbm.at[wid].at[1], sidx1, isem1)

        @pl.loop(0, NCH // 2)
        def _(g):
            for b in range(2):
                j = g * 2 + b
                nb = 1 - b

                @pl.when(j + 1 < NCH)
                def _():
                    pltpu.make_async_copy(
                        src_hbm.at[wid].at[0], sidx[nb], isem[nb]).wait()
                    pltpu.async_copy(hp_hbm.at[sidx[nb]], rows[nb], gsem[nb])

                pltpu.make_async_copy(
                    hp_hbm.at[pl.ds(0, CB)], rows[b], gsem[b]).wait()

                @pl.when(j + 2 < NCH)
                def _():
                    pltpu.async_copy(
                        src_hbm.at[wid].at[j + 2], sidx[b], isem[b])

                pltpu.sync_copy(rows[b], agg_sh.at[didx.at[j]], add=True)

        plsc.subcore_barrier()
        pltpu.sync_copy(agg_sh.at[pl.ds(lo, RPT)],
                        out_hbm.at[c].at[pl.ds(lo, RPT)])

    return deg_kernel, agg_kernel


def _deg_call(dstp, zerosD, onesD):
    return _sc_kernels()[0](dstp, zerosD, onesD)


def _agg_call(hp, srcp, dstp, zerosD):
    return _sc_kernels()[1](hp, srcp, dstp, zerosD)


def _mm1_body(x_ref, w_ref, deg_ref, hp_ref, dinv_ref):
    dsum = deg_ref[0][:, 0:1] + deg_ref[1][:, 0:1] + 1.0
    dinv = jnp.broadcast_to(lax.rsqrt(dsum), (NPAD, D))
    h = jnp.dot(x_ref[...], w_ref[...], preferred_element_type=jnp.float32)
    hp_ref[...] = h * dinv
    dinv_ref[...] = dinv


_mm1_call = pl.pallas_call(
    _mm1_body,
    out_shape=[
        jax.ShapeDtypeStruct((NPAD, D), jnp.float32),
        jax.ShapeDtypeStruct((NPAD, D), jnp.float32),
    ],
)


def _comb_body(agg_ref, hp_ref, dinv_ref, b_ref, w_ref, out_ref):
    tot = agg_ref[0] + agg_ref[1] + hp_ref[...]
    xn = jnp.maximum(tot * dinv_ref[...] + b_ref[...], 0.0)
    row = lax.broadcasted_iota(jnp.int32, (NPAD, D), 0)
    xn = jnp.where(row < N, xn, 0.0)
    out_ref[...] = jnp.dot(xn, w_ref[...],
                           preferred_element_type=jnp.float32) * dinv_ref[...]


_comb_call = pl.pallas_call(
    _comb_body,
    out_shape=jax.ShapeDtypeStruct((NPAD, D), jnp.float32),
)


def _head_body(agg_ref, hp_ref, dinv_ref, b3_ref, batch_ref, gamma_ref,
               beta_ref, wl1_ref, bl1_ref, wl2_ref, bl2_ref, out_ref):
    tot = agg_ref[0] + agg_ref[1] + hp_ref[...]
    h = jnp.maximum(tot * dinv_ref[...] + b3_ref[...], 0.0)
    bat = batch_ref[...]
    gids = lax.broadcasted_iota(jnp.int32, (NG, NPAD), 0)
    P = (bat == gids).astype(jnp.float32)
    sums = jnp.dot(P, h, preferred_element_type=jnp.float32)
    cnt = jnp.sum(P, axis=1, keepdims=True)
    pooled = sums / jnp.maximum(cnt, 1.0)
    mu = jnp.mean(pooled, axis=0, keepdims=True)
    var = jnp.mean((pooled - mu) * (pooled - mu), axis=0, keepdims=True)
    hn = (pooled - mu) * lax.rsqrt(var + 1e-5) * gamma_ref[...] + beta_ref[...]
    h1 = jnp.dot(hn, wl1_ref[...], preferred_element_type=jnp.float32)
    h1 = h1 + bl1_ref[...]
    logits = jnp.dot(h1, wl2_ref[...], preferred_element_type=jnp.float32)
    logits = logits + bl2_ref[...]
    z = logits - jnp.max(logits, axis=-1, keepdims=True)
    ez = jnp.exp(z)
    out_ref[...] = ez / jnp.sum(ez, axis=-1, keepdims=True)


_head_call = pl.pallas_call(
    _head_body,
    out_shape=jax.ShapeDtypeStruct((NG, D), jnp.float32),
)


def kernel(x, edge_index, batch, batch_size, W1, b1, W2, b2, W3, b3,
           gamma, beta, Wl1, bl1, Wl2, bl2):
    xp = jnp.pad(x.astype(jnp.float32), ((0, NPAD - N), (0, 0)))
    src = edge_index[0]
    dst = edge_index[1]
    fill = jnp.full((EP - E,), NPAD - 1, jnp.int32)
    srcp = jnp.concatenate([src.astype(jnp.int32), fill]).reshape(NW, NCH, CB)
    dstp = jnp.concatenate([dst.astype(jnp.int32), fill]).reshape(NW, NCH, CB)

    zerosD = jnp.zeros((NPAD, D), jnp.float32)
    onesD = jnp.ones((CB, D), jnp.float32)

    deg = _deg_call(dstp, zerosD, onesD)
    hp1, dinv = _mm1_call(xp, W1, deg)

    agg1 = _agg_call(hp1, srcp, dstp, zerosD)
    hp2 = _comb_call(agg1, hp1, dinv, b1.reshape(1, D), W2)
    agg2 = _agg_call(hp2, srcp, dstp, zerosD)
    hp3 = _comb_call(agg2, hp2, dinv, b2.reshape(1, D), W3)
    agg3 = _agg_call(hp3, srcp, dstp, zerosD)

    batchp = jnp.concatenate(
        [batch.astype(jnp.int32), jnp.full((NPAD - N,), NG, jnp.int32)]
    ).reshape(1, NPAD)
    wl2p = jnp.concatenate(
        [Wl2, jnp.zeros((D, D - NCLS), jnp.float32)], axis=1)
    bl2p = jnp.concatenate(
        [bl2, jnp.full((D - NCLS,), -1e9, jnp.float32)]).reshape(1, D)

    probs = _head_call(agg3, hp3, dinv, b3.reshape(1, D), batchp,
                       gamma.reshape(1, D), beta.reshape(1, D),
                       Wl1, bl1.reshape(1, D), wl2p, bl2p)
    return probs[:, :NCLS]


# speedup vs baseline: 2.0431x; 2.0431x over previous
import functools

import jax
import jax.numpy as jnp
from jax import lax
from jax.experimental import pallas as pl
from jax.experimental.pallas import tpu as pltpu
from jax.experimental.pallas import tpu_sc as plsc

N = 10000
E = 320000
D = 128
NG = 64
NCLS = 10

NC = 2
NS = 16
NW = NC * NS
CB = 80
NCH = 126
EP = NW * NCH * CB
NPAD = 10112
RPT = NPAD // NS
NROW = 80


@functools.lru_cache(maxsize=1)
def _sc_kernels():
    mesh = plsc.VectorSubcoreMesh(
        core_axis_name="c", subcore_axis_name="s",
        num_cores=NC, num_subcores=NS)

    def _unpack_dst(cidx, j, out_ref):
        for k in range(CB // 16):
            v = cidx[j, pl.ds(k * 16, 16)]
            out_ref[pl.ds(k * 16, 16)] = lax.shift_right_logical(v, 14)

    def _unpack_src(cidx, j, out_ref):
        for k in range(CB // 16):
            v = cidx[j, pl.ds(k * 16, 16)]
            out_ref[pl.ds(k * 16, 16)] = lax.bitwise_and(v, 16383)

    @functools.partial(
        pl.kernel,
        out_type=jax.ShapeDtypeStruct((NC, NPAD, D), jnp.float32),
        mesh=mesh,
        scratch_types=[
            pltpu.VMEM((NCH, CB), jnp.int32),
            [pltpu.VMEM((CB,), jnp.int32)] * 6,
            pltpu.VMEM((CB, D), jnp.float32),
            pltpu.VMEM_SHARED((NPAD, D), jnp.float32),
            pltpu.SemaphoreType.DMA,
        ],
    )
    def deg_kernel(cidx_hbm, zeros_hbm, ones_hbm, out_hbm, cidx, didx, ones_v,
                   deg_sh, sem):
        c = lax.axis_index("c")
        s = lax.axis_index("s")
        wid = c * NS + s
        lo = s * RPT
        pltpu.sync_copy(cidx_hbm.at[wid], cidx)
        pltpu.sync_copy(zeros_hbm.at[pl.ds(lo, RPT)], deg_sh.at[pl.ds(lo, RPT)])
        pltpu.sync_copy(ones_hbm, ones_v)
        plsc.subcore_barrier()

        @pl.loop(0, NCH // 6)
        def _(g):
            for b in range(6):
                _unpack_dst(cidx, g * 6 + b, didx[b])
                pltpu.async_copy(ones_v, deg_sh.at[didx[b]], sem, add=True)
            for b in range(6):
                pltpu.make_async_copy(ones_hbm, ones_v, sem).wait()

        plsc.subcore_barrier()
        pltpu.sync_copy(deg_sh.at[pl.ds(lo, RPT)],
                        out_hbm.at[c].at[pl.ds(lo, RPT)])

    @functools.partial(
        pl.kernel,
        out_type=jax.ShapeDtypeStruct((NC, NPAD, D), jnp.float32),
        mesh=mesh,
        scratch_types=[
            pltpu.VMEM((NCH, CB), jnp.int32),
            [pltpu.VMEM((CB,), jnp.int32)] * 3,
            pltpu.VMEM((CB,), jnp.int32),
            [pltpu.VMEM((CB, D), jnp.float32)] * 3,
            pltpu.VMEM_SHARED((NPAD, D), jnp.float32),
            [pltpu.SemaphoreType.DMA] * 3,
        ],
    )
    def agg_kernel(hp_hbm, cidx_hbm, zeros_hbm, out_hbm,
                   cidx, sidx, didx, rows, agg_sh, gsem):
        c = lax.axis_index("c")
        s = lax.axis_index("s")
        wid = c * NS + s
        lo = s * RPT

        pltpu.sync_copy(cidx_hbm.at[wid], cidx)
        pltpu.sync_copy(zeros_hbm.at[pl.ds(lo, RPT)], agg_sh.at[pl.ds(lo, RPT)])
        plsc.subcore_barrier()

        for b in range(3):
            _unpack_src(cidx, b, sidx[b])
            pltpu.async_copy(hp_hbm.at[sidx[b]], rows[b], gsem[b])

        @pl.loop(0, NCH // 3)
        def _(g):
            for r in range(3):
                j = g * 3 + r

                pltpu.make_async_copy(
                    hp_hbm.at[pl.ds(0, CB)], rows[r], gsem[r]).wait()

                _unpack_dst(cidx, j, didx)
                pltpu.sync_copy(rows[r], agg_sh.at[didx], add=True)

                @pl.when(j + 3 < NCH)
                def _():
                    _unpack_src(cidx, j + 3, sidx[r])
                    pltpu.async_copy(hp_hbm.at[sidx[r]], rows[r], gsem[r])

        plsc.subcore_barrier()
        pltpu.sync_copy(agg_sh.at[pl.ds(lo, RPT)],
                        out_hbm.at[c].at[pl.ds(lo, RPT)])

    return deg_kernel, agg_kernel


def _deg_call(cidxp, zerosD, onesD):
    return _sc_kernels()[0](cidxp, zerosD, onesD)


def _agg_call(hp, cidxp, zerosD):
    return _sc_kernels()[1](hp, cidxp, zerosD)


def _mm1_body(x_ref, w_ref, deg_ref, hp_ref, dinv_ref):
    dsum = deg_ref[0][:, 0:1] + deg_ref[1][:, 0:1] + 1.0
    dinv = jnp.broadcast_to(lax.rsqrt(dsum), (NPAD, D))
    h = jnp.dot(x_ref[...], w_ref[...], preferred_element_type=jnp.float32)
    hp_ref[...] = h * dinv
    dinv_ref[...] = dinv


_mm1_call = pl.pallas_call(
    _mm1_body,
    out_shape=[
        jax.ShapeDtypeStruct((NPAD, D), jnp.float32),
        jax.ShapeDtypeStruct((NPAD, D), jnp.float32),
    ],
)


def _comb_body(agg_ref, hp_ref, dinv_ref, b_ref, w_ref, out_ref):
    tot = agg_ref[0] + agg_ref[1] + hp_ref[...]
    xn = jnp.maximum(tot * dinv_ref[...] + b_ref[...], 0.0)
    row = lax.broadcasted_iota(jnp.int32, (NPAD, D), 0)
    xn = jnp.where(row < N, xn, 0.0)
    out_ref[...] = jnp.dot(xn, w_ref[...],
                           preferred_element_type=jnp.float32) * dinv_ref[...]


_comb_call = pl.pallas_call(
    _comb_body,
    out_shape=jax.ShapeDtypeStruct((NPAD, D), jnp.float32),
)


def _head_body(agg_ref, hp_ref, dinv_ref, b3_ref, batch_ref, gamma_ref,
               beta_ref, wl1_ref, bl1_ref, wl2_ref, bl2_ref, out_ref):
    tot = agg_ref[0] + agg_ref[1] + hp_ref[...]
    h = jnp.maximum(tot * dinv_ref[...] + b3_ref[...], 0.0)
    bat = batch_ref[...]
    gids = lax.broadcasted_iota(jnp.int32, (NG, NPAD), 0)
    P = (bat == gids).astype(jnp.float32)
    sums = jnp.dot(P, h, preferred_element_type=jnp.float32)
    cnt = jnp.sum(P, axis=1, keepdims=True)
    pooled = sums / jnp.maximum(cnt, 1.0)
    mu = jnp.mean(pooled, axis=0, keepdims=True)
    var = jnp.mean((pooled - mu) * (pooled - mu), axis=0, keepdims=True)
    hn = (pooled - mu) * lax.rsqrt(var + 1e-5) * gamma_ref[...] + beta_ref[...]
    h1 = jnp.dot(hn, wl1_ref[...], preferred_element_type=jnp.float32)
    h1 = h1 + bl1_ref[...]
    logits = jnp.dot(h1, wl2_ref[...], preferred_element_type=jnp.float32)
    logits = logits + bl2_ref[...]
    z = logits - jnp.max(logits, axis=-1, keepdims=True)
    ez = jnp.exp(z)
    out_ref[...] = ez / jnp.sum(ez, axis=-1, keepdims=True)


_head_call = pl.pallas_call(
    _head_body,
    out_shape=jax.ShapeDtypeStruct((NG, D), jnp.float32),
)


def kernel(x, edge_index, batch, batch_size, W1, b1, W2, b2, W3, b3,
           gamma, beta, Wl1, bl1, Wl2, bl2):
    xp = jnp.pad(x.astype(jnp.float32), ((0, NPAD - N), (0, 0)))
    src = edge_index[0]
    dst = edge_index[1]
    fill = jnp.full((EP - E,), NPAD - 1, jnp.int32)
    srcp = jnp.concatenate([src.astype(jnp.int32), fill])
    dstp = jnp.concatenate([dst.astype(jnp.int32), fill])
    cidxp = (srcp | (dstp << 14)).reshape(NW, NCH, CB)

    zerosD = jnp.zeros((NPAD, D), jnp.float32)
    onesD = jnp.ones((CB, D), jnp.float32)

    deg = _deg_call(cidxp, zerosD, onesD)
    hp1, dinv = _mm1_call(xp, W1, deg)

    agg1 = _agg_call(hp1, cidxp, zerosD)
    hp2 = _comb_call(agg1, hp1, dinv, b1.reshape(1, D), W2)
    agg2 = _agg_call(hp2, cidxp, zerosD)
    hp3 = _comb_call(agg2, hp2, dinv, b2.reshape(1, D), W3)
    agg3 = _agg_call(hp3, cidxp, zerosD)

    batchp = jnp.concatenate(
        [batch.astype(jnp.int32), jnp.full((NPAD - N,), NG, jnp.int32)]
    ).reshape(1, NPAD)
    wl2p = jnp.concatenate(
        [Wl2, jnp.zeros((D, D - NCLS), jnp.float32)], axis=1)
    bl2p = jnp.concatenate(
        [bl2, jnp.full((D - NCLS,), -1e9, jnp.float32)]).reshape(1, D)

    probs = _head_call(agg3, hp3, dinv, b3.reshape(1, D), batchp,
                       gamma.reshape(1, D), beta.reshape(1, D),
                       Wl1, bl1.reshape(1, D), wl2p, bl2p)
    return probs[:, :NCLS]


# speedup vs baseline: 2.0584x; 1.0075x over previous
import functools

import jax
import jax.numpy as jnp
from jax import lax
from jax.experimental import pallas as pl
from jax.experimental.pallas import tpu as pltpu
from jax.experimental.pallas import tpu_sc as plsc

N = 10000
E = 320000
D = 128
NG = 64
NCLS = 10

NC = 2
NS = 16
NW = NC * NS
CB = 80
NCH = 126
EP = NW * NCH * CB
NPAD = 10112
RPT = NPAD // NS
NROW = 80


@functools.lru_cache(maxsize=1)
def _sc_kernels():
    mesh = plsc.VectorSubcoreMesh(
        core_axis_name="c", subcore_axis_name="s",
        num_cores=NC, num_subcores=NS)

    def _unpack_dst(cidx, j, out_ref):
        for k in range(CB // 16):
            v = cidx[j, pl.ds(k * 16, 16)]
            out_ref[pl.ds(k * 16, 16)] = lax.shift_right_logical(v, 14)

    def _unpack_src(cidx, j, out_ref):
        for k in range(CB // 16):
            v = cidx[j, pl.ds(k * 16, 16)]
            out_ref[pl.ds(k * 16, 16)] = lax.bitwise_and(v, 16383)

    @functools.partial(
        pl.kernel,
        out_type=jax.ShapeDtypeStruct((NC, NPAD, D), jnp.float32),
        mesh=mesh,
        scratch_types=[
            pltpu.VMEM((NCH, CB), jnp.int32),
            [pltpu.VMEM((CB,), jnp.int32)] * 6,
            pltpu.VMEM((CB, D), jnp.float32),
            pltpu.VMEM_SHARED((NPAD, D), jnp.float32),
            pltpu.SemaphoreType.DMA,
        ],
    )
    def deg_kernel(cidx_hbm, zeros_hbm, ones_hbm, out_hbm, cidx, didx, ones_v,
                   deg_sh, sem):
        c = lax.axis_index("c")
        s = lax.axis_index("s")
        wid = c * NS + s
        lo = s * RPT
        pltpu.sync_copy(cidx_hbm.at[wid], cidx)
        pltpu.sync_copy(zeros_hbm.at[pl.ds(lo, RPT)], deg_sh.at[pl.ds(lo, RPT)])
        pltpu.sync_copy(ones_hbm, ones_v)
        plsc.subcore_barrier()

        @pl.loop(0, NCH // 6)
        def _(g):
            for b in range(6):
                _unpack_dst(cidx, g * 6 + b, didx[b])
                pltpu.async_copy(ones_v, deg_sh.at[didx[b]], sem, add=True)
            for b in range(6):
                pltpu.make_async_copy(ones_hbm, ones_v, sem).wait()

        plsc.subcore_barrier()
        pltpu.sync_copy(deg_sh.at[pl.ds(lo, RPT)],
                        out_hbm.at[c].at[pl.ds(lo, RPT)])

    @functools.partial(
        pl.kernel,
        out_type=jax.ShapeDtypeStruct((NC, NPAD, D), jnp.float32),
        mesh=mesh,
        scratch_types=[
            pltpu.VMEM((NCH, CB), jnp.int32),
            [pltpu.VMEM((CB,), jnp.int32)] * 3,
            pltpu.VMEM((CB,), jnp.int32),
            [pltpu.VMEM((CB, D), jnp.float32)] * 3,
            pltpu.VMEM_SHARED((NPAD, D), jnp.float32),
            [pltpu.SemaphoreType.DMA] * 3,
        ],
    )
    def agg_kernel(hp_hbm, cidx_hbm, zeros_hbm, out_hbm,
                   cidx, sidx, didx, rows, agg_sh, gsem):
        c = lax.axis_index("c")
        s = lax.axis_index("s")
        wid = c * NS + s
        lo = s * RPT

        pltpu.sync_copy(cidx_hbm.at[wid], cidx)

        for b in range(3):
            _unpack_src(cidx, b, sidx[b])
            pltpu.async_copy(hp_hbm.at[sidx[b]], rows[b], gsem[b])

        pltpu.sync_copy(zeros_hbm.at[pl.ds(lo, RPT)], agg_sh.at[pl.ds(lo, RPT)])
        plsc.subcore_barrier()

        @pl.loop(0, NCH // 3)
        def _(g):
            for r in range(3):
                j = g * 3 + r

                pltpu.make_async_copy(
                    hp_hbm.at[pl.ds(0, CB)], rows[r], gsem[r]).wait()

                _unpack_dst(cidx, j, didx)
                pltpu.sync_copy(rows[r], agg_sh.at[didx], add=True)

                @pl.when(j + 3 < NCH)
                def _():
                    _unpack_src(cidx, j + 3, sidx[r])
                    pltpu.async_copy(hp_hbm.at[sidx[r]], rows[r], gsem[r])

        plsc.subcore_barrier()
        pltpu.sync_copy(agg_sh.at[pl.ds(lo, RPT)],
                        out_hbm.at[c].at[pl.ds(lo, RPT)])

    return deg_kernel, agg_kernel


def _deg_call(cidxp, zerosD, onesD):
    return _sc_kernels()[0](cidxp, zerosD, onesD)


def _agg_call(hp, cidxp, zerosD):
    return _sc_kernels()[1](hp, cidxp, zerosD)


def _mm1_body(x_ref, w_ref, deg_ref, hp_ref, dinv_ref):
    dsum = deg_ref[0][:, 0:1] + deg_ref[1][:, 0:1] + 1.0
    dinv = jnp.broadcast_to(lax.rsqrt(dsum), (NPAD, D))
    h = jnp.dot(x_ref[...], w_ref[...], preferred_element_type=jnp.float32)
    hp_ref[...] = h * dinv
    dinv_ref[...] = dinv


_mm1_call = pl.pallas_call(
    _mm1_body,
    out_shape=[
        jax.ShapeDtypeStruct((NPAD, D), jnp.float32),
        jax.ShapeDtypeStruct((NPAD, D), jnp.float32),
    ],
)


def _comb_body(agg_ref, hp_ref, dinv_ref, b_ref, w_ref, out_ref):
    tot = agg_ref[0] + agg_ref[1] + hp_ref[...]
    xn = jnp.maximum(tot * dinv_ref[...] + b_ref[...], 0.0)
    row = lax.broadcasted_iota(jnp.int32, (NPAD, D), 0)
    xn = jnp.where(row < N, xn, 0.0)
    out_ref[...] = jnp.dot(xn, w_ref[...],
                           preferred_element_type=jnp.float32) * dinv_ref[...]


_comb_call = pl.pallas_call(
    _comb_body,
    out_shape=jax.ShapeDtypeStruct((NPAD, D), jnp.float32),
)


def _head_body(agg_ref, hp_ref, dinv_ref, b3_ref, batch_ref, gamma_ref,
               beta_ref, wl1_ref, bl1_ref, wl2_ref, bl2_ref, out_ref):
    tot = agg_ref[0] + agg_ref[1] + hp_ref[...]
    h = jnp.maximum(tot * dinv_ref[...] + b3_ref[...], 0.0)
    bat = batch_ref[...]
    gids = lax.broadcasted_iota(jnp.int32, (NG, NPAD), 0)
    P = (bat == gids).astype(jnp.float32)
    sums = jnp.dot(P, h, preferred_element_type=jnp.float32)
    cnt = jnp.sum(P, axis=1, keepdims=True)
    pooled = sums / jnp.maximum(cnt, 1.0)
    mu = jnp.mean(pooled, axis=0, keepdims=True)
    var = jnp.mean((pooled - mu) * (pooled - mu), axis=0, keepdims=True)
    hn = (pooled - mu) * lax.rsqrt(var + 1e-5) * gamma_ref[...] + beta_ref[...]
    h1 = jnp.dot(hn, wl1_ref[...], preferred_element_type=jnp.float32)
    h1 = h1 + bl1_ref[...]
    logits = jnp.dot(h1, wl2_ref[...], preferred_element_type=jnp.float32)
    logits = logits + bl2_ref[...]
    z = logits - jnp.max(logits, axis=-1, keepdims=True)
    ez = jnp.exp(z)
    out_ref[...] = ez / jnp.sum(ez, axis=-1, keepdims=True)


_head_call = pl.pallas_call(
    _head_body,
    out_shape=jax.ShapeDtypeStruct((NG, D), jnp.float32),
)


def kernel(x, edge_index, batch, batch_size, W1, b1, W2, b2, W3, b3,
           gamma, beta, Wl1, bl1, Wl2, bl2):
    xp = jnp.pad(x.astype(jnp.float32), ((0, NPAD - N), (0, 0)))
    src = edge_index[0]
    dst = edge_index[1]
    fill = jnp.full((EP - E,), NPAD - 1, jnp.int32)
    srcp = jnp.concatenate([src.astype(jnp.int32), fill])
    dstp = jnp.concatenate([dst.astype(jnp.int32), fill])
    cidxp = (srcp | (dstp << 14)).reshape(NW, NCH, CB)

    zerosD = jnp.zeros((NPAD, D), jnp.float32)
    onesD = jnp.ones((CB, D), jnp.float32)

    deg = _deg_call(cidxp, zerosD, onesD)
    hp1, dinv = _mm1_call(xp, W1, deg)

    agg1 = _agg_call(hp1, cidxp, zerosD)
    hp2 = _comb_call(agg1, hp1, dinv, b1.reshape(1, D), W2)
    agg2 = _agg_call(hp2, cidxp, zerosD)
    hp3 = _comb_call(agg2, hp2, dinv, b2.reshape(1, D), W3)
    agg3 = _agg_call(hp3, cidxp, zerosD)

    batchp = jnp.concatenate(
        [batch.astype(jnp.int32), jnp.full((NPAD - N,), NG, jnp.int32)]
    ).reshape(1, NPAD)
    wl2p = jnp.concatenate(
        [Wl2, jnp.zeros((D, D - NCLS), jnp.float32)], axis=1)
    bl2p = jnp.concatenate(
        [bl2, jnp.full((D - NCLS,), -1e9, jnp.float32)]).reshape(1, D)

    probs = _head_call(agg3, hp3, dinv, b3.reshape(1, D), batchp,
                       gamma.reshape(1, D), beta.reshape(1, D),
                       Wl1, bl1.reshape(1, D), wl2p, bl2p)
    return probs[:, :NCLS]
